# baseline (speedup 1.0000x reference)
import functools
import math

import jax
import jax.numpy as jnp
from jax import lax
from jax.experimental import pallas as pl
from jax.experimental.pallas import tpu as pltpu
from jax.experimental.pallas import tpu_sc as plsc

D_MODEL = 768
SEQ_LEN = 4096
BATCH = 4
N_FLAT = SEQ_LEN * BATCH

NUM_WORKERS = 32
PER_WORKER = N_FLAT // NUM_WORKERS
POS_PER_WORKER = PER_WORKER // BATCH
CHUNK = 32
NCHUNK = PER_WORKER // CHUNK
POS_PER_CHUNK = CHUNK // BATCH
NSLOT = 4
LEAD = 2
LANES = 16
VREGS_PER_ROW = D_MODEL // LANES
SCALE = math.sqrt(D_MODEL)


def _make_kernel():
    mesh = plsc.VectorSubcoreMesh(core_axis_name="c", subcore_axis_name="s")

    @functools.partial(
        pl.kernel,
        mesh=mesh,
        out_type=jax.ShapeDtypeStruct((SEQ_LEN, BATCH, D_MODEL), jnp.float32),
        scratch_types=[
            pltpu.VMEM((NCHUNK, CHUNK), jnp.int32),
            pltpu.VMEM((NSLOT, POS_PER_CHUNK, 1, D_MODEL), jnp.float32),
            pltpu.VMEM((NSLOT, CHUNK, D_MODEL), jnp.float32),
            pltpu.SemaphoreType.DMA((NSLOT,)),
            pltpu.SemaphoreType.DMA((NSLOT,)),
        ],
    )
    def k(x_hbm, table_hbm, pe_hbm, out_hbm, idx_v, pe_v, buf_v, isem, osem):
        cid = lax.axis_index("c")
        sid = lax.axis_index("s")
        wid = sid * 2 + cid
        pos0 = wid * POS_PER_WORKER
        out_flat = out_hbm.reshape(N_FLAT, D_MODEL)

        pltpu.sync_copy(x_hbm.at[wid], idx_v)

        def fire_in(c):
            s = c % NSLOT
            g = pltpu.async_copy(table_hbm.at[idx_v.at[c]], buf_v.at[s],
                                 isem.at[s])
            p = pltpu.async_copy(
                pe_hbm.at[pl.ds(pos0 + c * POS_PER_CHUNK, POS_PER_CHUNK)],
                pe_v.at[s], isem.at[s])
            return g, p

        def make_store(c):
            s = c % NSLOT
            return pltpu.make_async_copy(
                buf_v.at[s],
                out_flat.at[pl.ds(wid * PER_WORKER + c * CHUNK, CHUNK)],
                osem.at[s])

        def compute(s):
            @plsc.parallel_loop(0, POS_PER_CHUNK)
            def p_body(p):
                base = p * BATCH
                for j in range(VREGS_PER_ROW):
                    off = j * LANES
                    pvec = pe_v[s, p, 0, pl.ds(off, LANES)]
                    for b in range(BATCH):
                        buf_v[s, base + b, pl.ds(off, LANES)] = (
                            buf_v[s, base + b, pl.ds(off, LANES)] * SCALE
                            + pvec)

        for c in range(LEAD):
            fire_in(c)

        def chunk_body(c, _):
            s = c % NSLOT

            @pl.when(c + LEAD < NCHUNK)
            def _():
                @pl.when(c >= LEAD)
                def _():
                    make_store(lax.max(c - LEAD, 0)).wait()

                fire_in(c + LEAD)

            g = pltpu.make_async_copy(table_hbm.at[idx_v.at[c]], buf_v.at[s],
                                      isem.at[s])
            p = pltpu.make_async_copy(
                pe_hbm.at[pl.ds(pos0 + c * POS_PER_CHUNK, POS_PER_CHUNK)],
                pe_v.at[s], isem.at[s])
            g.wait()
            p.wait()
            compute(s)
            pltpu.async_copy(
                buf_v.at[s],
                out_flat.at[pl.ds(wid * PER_WORKER + c * CHUNK, CHUNK)],
                osem.at[s])
            return 0

        lax.fori_loop(0, NCHUNK, chunk_body, 0)

        for c in range(NCHUNK - 2 * LEAD, NCHUNK):
            make_store(c).wait()

    return k


_sc_kernel = _make_kernel()


def kernel(x, table, pe):
    xf = jnp.asarray(x, jnp.int32).reshape(NUM_WORKERS, NCHUNK, CHUNK)
    return _sc_kernel(xf, table, pe)


# speedup vs baseline: 1.0161x; 1.0161x over previous
import functools
import math

import jax
import jax.numpy as jnp
from jax import lax
from jax.experimental import pallas as pl
from jax.experimental.pallas import tpu as pltpu
from jax.experimental.pallas import tpu_sc as plsc

D_MODEL = 768
SEQ_LEN = 4096
BATCH = 4
N_FLAT = SEQ_LEN * BATCH

NUM_WORKERS = 32
PER_WORKER = N_FLAT // NUM_WORKERS
POS_PER_WORKER = PER_WORKER // BATCH
CHUNK = 16
NCHUNK = PER_WORKER // CHUNK
POS_PER_CHUNK = CHUNK // BATCH
NSLOT = 8
LEAD = 4
LANES = 16
VREGS_PER_ROW = D_MODEL // LANES
SCALE = math.sqrt(D_MODEL)


def _make_kernel():
    mesh = plsc.VectorSubcoreMesh(core_axis_name="c", subcore_axis_name="s")

    @functools.partial(
        pl.kernel,
        mesh=mesh,
        out_type=jax.ShapeDtypeStruct((SEQ_LEN, BATCH, D_MODEL), jnp.float32),
        scratch_types=[
            pltpu.VMEM((NCHUNK, CHUNK), jnp.int32),
            pltpu.VMEM((NSLOT, POS_PER_CHUNK, 1, D_MODEL), jnp.float32),
            pltpu.VMEM((NSLOT, CHUNK, D_MODEL), jnp.float32),
            pltpu.SemaphoreType.DMA((NSLOT,)),
            pltpu.SemaphoreType.DMA((NSLOT,)),
        ],
    )
    def k(x_hbm, table_hbm, pe_hbm, out_hbm, idx_v, pe_v, buf_v, isem, osem):
        cid = lax.axis_index("c")
        sid = lax.axis_index("s")
        wid = sid * 2 + cid
        pos0 = wid * POS_PER_WORKER
        out_flat = out_hbm.reshape(N_FLAT, D_MODEL)

        pltpu.sync_copy(x_hbm.at[wid], idx_v)

        def fire_in(c):
            s = c % NSLOT
            g = pltpu.async_copy(table_hbm.at[idx_v.at[c]], buf_v.at[s],
                                 isem.at[s])
            p = pltpu.async_copy(
                pe_hbm.at[pl.ds(pos0 + c * POS_PER_CHUNK, POS_PER_CHUNK)],
                pe_v.at[s], isem.at[s])
            return g, p

        def make_store(c):
            s = c % NSLOT
            return pltpu.make_async_copy(
                buf_v.at[s],
                out_flat.at[pl.ds(wid * PER_WORKER + c * CHUNK, CHUNK)],
                osem.at[s])

        def compute(s):
            @plsc.parallel_loop(0, POS_PER_CHUNK)
            def p_body(p):
                base = p * BATCH
                for j in range(VREGS_PER_ROW):
                    off = j * LANES
                    pvec = pe_v[s, p, 0, pl.ds(off, LANES)]
                    for b in range(BATCH):
                        buf_v[s, base + b, pl.ds(off, LANES)] = (
                            buf_v[s, base + b, pl.ds(off, LANES)] * SCALE
                            + pvec)

        for c in range(LEAD):
            fire_in(c)

        def chunk_body(c, _):
            s = c % NSLOT

            @pl.when(c + LEAD < NCHUNK)
            def _():
                @pl.when(c >= LEAD)
                def _():
                    make_store(lax.max(c - LEAD, 0)).wait()

                fire_in(c + LEAD)

            g = pltpu.make_async_copy(table_hbm.at[idx_v.at[c]], buf_v.at[s],
                                      isem.at[s])
            p = pltpu.make_async_copy(
                pe_hbm.at[pl.ds(pos0 + c * POS_PER_CHUNK, POS_PER_CHUNK)],
                pe_v.at[s], isem.at[s])
            g.wait()
            p.wait()
            compute(s)
            plsc.subcore_barrier()
            pltpu.async_copy(
                buf_v.at[s],
                out_flat.at[pl.ds(wid * PER_WORKER + c * CHUNK, CHUNK)],
                osem.at[s])
            return 0

        lax.fori_loop(0, NCHUNK, chunk_body, 0)

        for c in range(NCHUNK - 2 * LEAD, NCHUNK):
            make_store(c).wait()

    return k


_sc_kernel = _make_kernel()


def kernel(x, table, pe):
    xf = jnp.asarray(x, jnp.int32).reshape(NUM_WORKERS, NCHUNK, CHUNK)
    return _sc_kernel(xf, table, pe)
